# Initial kernel scaffold; baseline (speedup 1.0000x reference)
import jax
import jax.numpy as jnp
from jax.experimental import pallas as pl


def kernel(x, edges, kernel, kernel_attention1, kernel_attention2, bias, training):
    raise NotImplementedError("write your pallas kernel here")


# speedup vs baseline: 61.6744x; 61.6744x over previous
import functools

import jax
import jax.numpy as jnp
from jax import lax
from jax.experimental import pallas as pl
from jax.experimental.pallas import tpu as pltpu
from jax.experimental.pallas import tpu_sc as plsc

N_NODES = 10000
N_PAD = 10240
E_TOTAL = 320000
D_IN = 128
NUM_HEADS = 8
HEAD_DIM = 16
FEAT = NUM_HEADS * HEAD_DIM

NUM_CORES = 2
NUM_SUBCORES = 16
NUM_WORKERS = NUM_CORES * NUM_SUBCORES
E_PER_W = E_TOTAL // NUM_WORKERS
B_EDGE = 80
NUM_BLOCKS = E_PER_W // B_EDGE
ROWS_PER_TILE = N_PAD // NUM_SUBCORES
ACC_CH = 128
NUM_CH = ROWS_PER_TILE // ACC_CH


def _proj_body(x_ref, w_ref, a_ref, xk_ref, ftab_ref):
    xk = jnp.dot(x_ref[...], w_ref[...], preferred_element_type=jnp.float32)
    xk_ref[...] = xk
    ftab_ref[...] = jnp.dot(xk, a_ref[...], preferred_element_type=jnp.float32)


def _project(x, w, a16):
    rb = 1000
    return pl.pallas_call(
        _proj_body,
        grid=(N_NODES // rb,),
        in_specs=[
            pl.BlockSpec((rb, D_IN), lambda i: (i, 0)),
            pl.BlockSpec((D_IN, FEAT), lambda i: (0, 0)),
            pl.BlockSpec((FEAT, 16), lambda i: (0, 0)),
        ],
        out_specs=[
            pl.BlockSpec((rb, FEAT), lambda i: (i, 0)),
            pl.BlockSpec((rb, 16), lambda i: (i, 0)),
        ],
        out_shape=[
            jax.ShapeDtypeStruct((N_NODES, FEAT), jnp.float32),
            jax.ShapeDtypeStruct((N_NODES, 16), jnp.float32),
        ],
    )(x, w, a16)


def _edge_body(xk, ftab, src, dst, acc_out, s_out,
               idx_s, idx_t, idx_t2, rows_x, rows_t, rows_t2, prow,
               acc_sh, sacc_sh, sem_x, sem_t, sem_t2):
    cid = lax.axis_index("c")
    sid = lax.axis_index("s")
    wid = cid * NUM_SUBCORES + sid

    zero16 = jnp.zeros((16,), jnp.float32)

    def zrow(i, carry):
        for c in range(FEAT // 16):
            rows_x[i, pl.ds(c * 16, 16)] = zero16
        prow[i, :] = zero16
        return carry

    lax.fori_loop(0, B_EDGE, zrow, 0)

    base_row = sid * ROWS_PER_TILE
    for k in range(ROWS_PER_TILE // B_EDGE):
        pltpu.sync_copy(rows_x, acc_sh.at[pl.ds(base_row + k * B_EDGE, B_EDGE), :])
        pltpu.sync_copy(prow, sacc_sh.at[pl.ds(base_row + k * B_EDGE, B_EDGE), :])
    plsc.subcore_barrier()

    ebase = wid * E_PER_W
    lanes = lax.iota(jnp.int32, 16)
    head_mask = lanes < NUM_HEADS

    def block(b, carry):
        off = pl.multiple_of(ebase + b * B_EDGE, 8)
        pltpu.sync_copy(src.at[pl.ds(off, B_EDGE)], idx_s)
        pltpu.sync_copy(dst.at[pl.ds(off, B_EDGE)], idx_t)
        cp_x = pltpu.async_copy(xk.at[idx_s], rows_x, sem_x)
        cp_t = pltpu.async_copy(ftab.at[idx_t], rows_t, sem_t)
        pltpu.sync_copy(dst.at[idx_s], idx_t2)
        cp_t2 = pltpu.async_copy(ftab.at[idx_t2], rows_t2, sem_t2)
        cp_t.wait()
        cp_t2.wait()
        cp_x.wait()

        def edge(e, ecarry):
            z = rows_t[e, :] + rows_t2[e, :]
            s = jnp.maximum(z, 0.2 * z)
            p = jnp.where(head_mask, jnp.exp(s), 0.0)
            prow[e, :] = p
            for h in range(NUM_HEADS):
                wv = p[h]
                sl = pl.ds(h * HEAD_DIM, HEAD_DIM)
                rows_x[e, sl] = rows_x[e, sl] * wv
            return ecarry

        lax.fori_loop(0, B_EDGE, edge, 0)
        pltpu.sync_copy(rows_x, acc_sh.at[idx_t], add=True)
        pltpu.sync_copy(prow, sacc_sh.at[idx_t], add=True)
        return carry

    lax.fori_loop(0, NUM_BLOCKS, block, 0)
    plsc.subcore_barrier()

    for k in range(NUM_CH):
        r0 = base_row + k * ACC_CH
        pltpu.sync_copy(acc_sh.at[pl.ds(r0, ACC_CH), :],
                        acc_out.at[cid, pl.ds(r0, ACC_CH), :])
        pltpu.sync_copy(sacc_sh.at[pl.ds(r0, ACC_CH), :],
                        s_out.at[cid, pl.ds(r0, ACC_CH), :])


_edge_pass = functools.partial(
    pl.kernel,
    out_type=[
        jax.ShapeDtypeStruct((NUM_CORES, N_PAD, FEAT), jnp.float32),
        jax.ShapeDtypeStruct((NUM_CORES, N_PAD, 16), jnp.float32),
    ],
    mesh=plsc.VectorSubcoreMesh(
        core_axis_name="c", subcore_axis_name="s",
        num_cores=NUM_CORES, num_subcores=NUM_SUBCORES),
    compiler_params=pltpu.CompilerParams(use_tc_tiling_on_sc=False),
    scratch_types=[
        pltpu.VMEM((B_EDGE,), jnp.int32),
        pltpu.VMEM((B_EDGE,), jnp.int32),
        pltpu.VMEM((B_EDGE,), jnp.int32),
        pltpu.VMEM((B_EDGE, FEAT), jnp.float32),
        pltpu.VMEM((B_EDGE, 16), jnp.float32),
        pltpu.VMEM((B_EDGE, 16), jnp.float32),
        pltpu.VMEM((B_EDGE, 16), jnp.float32),
        pltpu.VMEM_SHARED((N_PAD, FEAT), jnp.float32),
        pltpu.VMEM_SHARED((N_PAD, 16), jnp.float32),
        pltpu.SemaphoreType.DMA,
        pltpu.SemaphoreType.DMA,
        pltpu.SemaphoreType.DMA,
    ],
)(_edge_body)


def _final_body(acc_ref, s_ref, rep_ref, bias_ref, o_ref):
    num = acc_ref[0] + acc_ref[1]
    den = s_ref[0] + s_ref[1]
    den_rep = jnp.dot(den, rep_ref[...], preferred_element_type=jnp.float32)
    y = num / (den_rep + 1e-7) + bias_ref[...]
    o_ref[...] = jnp.where(y > 0, y, jnp.exp(jnp.minimum(y, 0.0)) - 1.0)


def _finalize(acc, s, rep, bias2d):
    rb = 1000
    return pl.pallas_call(
        _final_body,
        grid=(N_NODES // rb,),
        in_specs=[
            pl.BlockSpec((NUM_CORES, rb, FEAT), lambda i: (0, i, 0)),
            pl.BlockSpec((NUM_CORES, rb, 16), lambda i: (0, i, 0)),
            pl.BlockSpec((16, FEAT), lambda i: (0, 0)),
            pl.BlockSpec((1, FEAT), lambda i: (0, 0)),
        ],
        out_specs=pl.BlockSpec((rb, FEAT), lambda i: (i, 0)),
        out_shape=jax.ShapeDtypeStruct((N_NODES, FEAT), jnp.float32),
    )(acc, s, rep, bias2d)


def kernel(x, edges, kernel, kernel_attention1, kernel_attention2, bias, training):
    del kernel_attention2, training
    src = edges[:, 0].astype(jnp.int32)
    dst = edges[:, 1].astype(jnp.int32)

    ka1 = kernel_attention1[0].astype(jnp.float32)
    a8 = (ka1[:, :, None] * jnp.eye(NUM_HEADS, dtype=jnp.float32)[:, None, :])
    a16 = jnp.concatenate(
        [a8.reshape(FEAT, NUM_HEADS),
         jnp.zeros((FEAT, NUM_HEADS), jnp.float32)], axis=1)

    rep = jnp.concatenate(
        [jnp.kron(jnp.eye(NUM_HEADS, dtype=jnp.float32),
                  jnp.ones((1, HEAD_DIM), jnp.float32)),
         jnp.zeros((NUM_HEADS, FEAT), jnp.float32)], axis=0)

    xk, ftab = _project(x, kernel.astype(jnp.float32), a16)
    acc, s = _edge_pass(xk, ftab, src, dst)
    return _finalize(acc, s, rep, bias.reshape(1, FEAT).astype(jnp.float32))


# speedup vs baseline: 87.5275x; 1.4192x over previous
import functools

import jax
import jax.numpy as jnp
from jax import lax
from jax.experimental import pallas as pl
from jax.experimental.pallas import tpu as pltpu
from jax.experimental.pallas import tpu_sc as plsc

N_NODES = 10000
N_PAD = 10240
E_TOTAL = 320000
D_IN = 128
NUM_HEADS = 8
HEAD_DIM = 16
FEAT = NUM_HEADS * HEAD_DIM

NUM_CORES = 2
NUM_SUBCORES = 16
NUM_WORKERS = NUM_CORES * NUM_SUBCORES
E_PER_W = E_TOTAL // NUM_WORKERS
B_EDGE = 80
NUM_BLOCKS = E_PER_W // B_EDGE
ROWS_PER_TILE = N_PAD // NUM_SUBCORES
ACC_CH = 128
NUM_CH = ROWS_PER_TILE // ACC_CH


def _proj_body(x_ref, w_ref, a_ref, xk_ref, ftab_ref):
    xk = jnp.dot(x_ref[...], w_ref[...], preferred_element_type=jnp.float32)
    xk_ref[...] = xk
    ftab_ref[...] = jnp.dot(xk, a_ref[...], preferred_element_type=jnp.float32)


def _project(x, w, a16):
    rb = 1000
    return pl.pallas_call(
        _proj_body,
        grid=(N_NODES // rb,),
        in_specs=[
            pl.BlockSpec((rb, D_IN), lambda i: (i, 0)),
            pl.BlockSpec((D_IN, FEAT), lambda i: (0, 0)),
            pl.BlockSpec((FEAT, 16), lambda i: (0, 0)),
        ],
        out_specs=[
            pl.BlockSpec((rb, FEAT), lambda i: (i, 0)),
            pl.BlockSpec((rb, 16), lambda i: (i, 0)),
        ],
        out_shape=[
            jax.ShapeDtypeStruct((N_NODES, FEAT), jnp.float32),
            jax.ShapeDtypeStruct((N_NODES, 16), jnp.float32),
        ],
    )(x, w, a16)


def _edge_body(xk, ftab, src, dst, acc_out, s_out,
               is0, it0, iu0, rx0, rt0, ru0, pr0,
               is1, it1, iu1, rx1, rt1, ru1, pr1,
               acc_sh, sacc_sh, sem_i, sem_g, sem_r0, sem_r1):
    cid = lax.axis_index("c")
    sid = lax.axis_index("s")
    wid = cid * NUM_SUBCORES + sid
    ebase = wid * E_PER_W
    LAST = NUM_BLOCKS - 1

    BUFS = ((is0, it0, iu0, rx0, rt0, ru0, pr0, sem_r0),
            (is1, it1, iu1, rx1, rt1, ru1, pr1, sem_r1))

    zero16 = jnp.zeros((16,), jnp.float32)

    def zrow(i, carry):
        for c in range(FEAT // 16):
            rx0[i, pl.ds(c * 16, 16)] = zero16
        pr0[i, :] = zero16
        return carry

    lax.fori_loop(0, B_EDGE, zrow, 0)
    base_row = sid * ROWS_PER_TILE
    for k in range(ROWS_PER_TILE // B_EDGE):
        pltpu.sync_copy(rx0, acc_sh.at[pl.ds(base_row + k * B_EDGE, B_EDGE), :])
        pltpu.sync_copy(pr0, sacc_sh.at[pl.ds(base_row + k * B_EDGE, B_EDGE), :])
    plsc.subcore_barrier()

    lanes = lax.iota(jnp.int32, 16)
    head_mask = lanes < NUM_HEADS

    def off(m):
        return pl.multiple_of(ebase + m * B_EDGE, 8)

    def start_idx(m, b):
        pltpu.async_copy(src.at[pl.ds(off(m), B_EDGE)], b[0], sem_i)
        pltpu.async_copy(dst.at[pl.ds(off(m), B_EDGE)], b[1], sem_i)

    def wait_idx(m, b):
        pltpu.make_async_copy(src.at[pl.ds(off(m), B_EDGE)], b[0], sem_i).wait()
        pltpu.make_async_copy(dst.at[pl.ds(off(m), B_EDGE)], b[1], sem_i).wait()

    def start_t2(b):
        pltpu.async_copy(dst.at[b[0]], b[2], sem_g)

    def wait_t2(b):
        pltpu.make_async_copy(dst.at[b[0]], b[2], sem_g).wait()

    def start_rows_xt(b):
        pltpu.async_copy(xk.at[b[0]], b[3], b[7])
        pltpu.async_copy(ftab.at[b[1]], b[4], b[7])

    def start_rows_u(b):
        pltpu.async_copy(ftab.at[b[2]], b[5], b[7])

    def wait_rows(b):
        pltpu.make_async_copy(xk.at[b[0]], b[3], b[7]).wait()
        pltpu.make_async_copy(ftab.at[b[1]], b[4], b[7]).wait()
        pltpu.make_async_copy(ftab.at[b[2]], b[5], b[7]).wait()

    def compute(b):
        rows_x, rows_t, rows_u, prow = b[3], b[4], b[5], b[6]

        def edge(e, ecarry):
            z = rows_t[e, :] + rows_u[e, :]
            s = jnp.maximum(z, 0.2 * z)
            p = jnp.where(head_mask, jnp.exp(s), 0.0)
            prow[e, :] = p
            for h in range(NUM_HEADS):
                wv = p[h]
                sl = pl.ds(h * HEAD_DIM, HEAD_DIM)
                rows_x[e, sl] = rows_x[e, sl] * wv
            return ecarry

        lax.fori_loop(0, B_EDGE, edge, 0)

    def scatter(b):
        pltpu.sync_copy(b[3], acc_sh.at[b[1]], add=True)
        pltpu.sync_copy(b[6], sacc_sh.at[b[1]], add=True)

    def process(m, par):
        cur, nxt = BUFS[par], BUFS[1 - par]

        @pl.when(m + 1 <= LAST)
        def _():
            wait_idx(m + 1, nxt)
            start_t2(nxt)
            start_rows_xt(nxt)

        wait_rows(cur)
        compute(cur)
        scatter(cur)

        @pl.when(m + 2 <= LAST)
        def _():
            start_idx(m + 2, cur)

        @pl.when(m + 1 <= LAST)
        def _():
            wait_t2(nxt)
            start_rows_u(nxt)

    start_idx(0, BUFS[0])
    wait_idx(0, BUFS[0])
    start_t2(BUFS[0])
    start_rows_xt(BUFS[0])
    start_idx(1, BUFS[1])
    wait_t2(BUFS[0])
    start_rows_u(BUFS[0])

    def outer(i, carry):
        process(2 * i, 0)
        process(2 * i + 1, 1)
        return carry

    lax.fori_loop(0, NUM_BLOCKS // 2, outer, 0)
    process(LAST, 0)
    plsc.subcore_barrier()

    for k in range(NUM_CH):
        r0 = base_row + k * ACC_CH
        pltpu.sync_copy(acc_sh.at[pl.ds(r0, ACC_CH), :],
                        acc_out.at[cid, pl.ds(r0, ACC_CH), :])
        pltpu.sync_copy(sacc_sh.at[pl.ds(r0, ACC_CH), :],
                        s_out.at[cid, pl.ds(r0, ACC_CH), :])

_edge_pass = functools.partial(
    pl.kernel,
    out_type=[
        jax.ShapeDtypeStruct((NUM_CORES, N_PAD, FEAT), jnp.float32),
        jax.ShapeDtypeStruct((NUM_CORES, N_PAD, 16), jnp.float32),
    ],
    mesh=plsc.VectorSubcoreMesh(
        core_axis_name="c", subcore_axis_name="s",
        num_cores=NUM_CORES, num_subcores=NUM_SUBCORES),
    compiler_params=pltpu.CompilerParams(use_tc_tiling_on_sc=False),
    scratch_types=[
        pltpu.VMEM((B_EDGE,), jnp.int32),
        pltpu.VMEM((B_EDGE,), jnp.int32),
        pltpu.VMEM((B_EDGE,), jnp.int32),
        pltpu.VMEM((B_EDGE, FEAT), jnp.float32),
        pltpu.VMEM((B_EDGE, 16), jnp.float32),
        pltpu.VMEM((B_EDGE, 16), jnp.float32),
        pltpu.VMEM((B_EDGE, 16), jnp.float32),
        pltpu.VMEM((B_EDGE,), jnp.int32),
        pltpu.VMEM((B_EDGE,), jnp.int32),
        pltpu.VMEM((B_EDGE,), jnp.int32),
        pltpu.VMEM((B_EDGE, FEAT), jnp.float32),
        pltpu.VMEM((B_EDGE, 16), jnp.float32),
        pltpu.VMEM((B_EDGE, 16), jnp.float32),
        pltpu.VMEM((B_EDGE, 16), jnp.float32),
        pltpu.VMEM_SHARED((N_PAD, FEAT), jnp.float32),
        pltpu.VMEM_SHARED((N_PAD, 16), jnp.float32),
        pltpu.SemaphoreType.DMA,
        pltpu.SemaphoreType.DMA,
        pltpu.SemaphoreType.DMA,
        pltpu.SemaphoreType.DMA,
    ],
)(_edge_body)


def _final_body(acc_ref, s_ref, rep_ref, bias_ref, o_ref):
    num = acc_ref[0] + acc_ref[1]
    den = s_ref[0] + s_ref[1]
    den_rep = jnp.dot(den, rep_ref[...], preferred_element_type=jnp.float32)
    y = num / (den_rep + 1e-7) + bias_ref[...]
    o_ref[...] = jnp.where(y > 0, y, jnp.exp(jnp.minimum(y, 0.0)) - 1.0)


def _finalize(acc, s, rep, bias2d):
    rb = 1000
    return pl.pallas_call(
        _final_body,
        grid=(N_NODES // rb,),
        in_specs=[
            pl.BlockSpec((NUM_CORES, rb, FEAT), lambda i: (0, i, 0)),
            pl.BlockSpec((NUM_CORES, rb, 16), lambda i: (0, i, 0)),
            pl.BlockSpec((16, FEAT), lambda i: (0, 0)),
            pl.BlockSpec((1, FEAT), lambda i: (0, 0)),
        ],
        out_specs=pl.BlockSpec((rb, FEAT), lambda i: (i, 0)),
        out_shape=jax.ShapeDtypeStruct((N_NODES, FEAT), jnp.float32),
    )(acc, s, rep, bias2d)


def kernel(x, edges, kernel, kernel_attention1, kernel_attention2, bias, training):
    del kernel_attention2, training
    src = edges[:, 0].astype(jnp.int32)
    dst = edges[:, 1].astype(jnp.int32)

    ka1 = kernel_attention1[0].astype(jnp.float32)
    a8 = (ka1[:, :, None] * jnp.eye(NUM_HEADS, dtype=jnp.float32)[:, None, :])
    a16 = jnp.concatenate(
        [a8.reshape(FEAT, NUM_HEADS),
         jnp.zeros((FEAT, NUM_HEADS), jnp.float32)], axis=1)

    rep = jnp.concatenate(
        [jnp.kron(jnp.eye(NUM_HEADS, dtype=jnp.float32),
                  jnp.ones((1, HEAD_DIM), jnp.float32)),
         jnp.zeros((NUM_HEADS, FEAT), jnp.float32)], axis=0)

    xk, ftab = _project(x, kernel.astype(jnp.float32), a16)
    acc, s = _edge_pass(xk, ftab, src, dst)
    return _finalize(acc, s, rep, bias.reshape(1, FEAT).astype(jnp.float32))


# speedup vs baseline: 129.0193x; 1.4740x over previous
import functools

import jax
import jax.numpy as jnp
from jax import lax
from jax.experimental import pallas as pl
from jax.experimental.pallas import tpu as pltpu
from jax.experimental.pallas import tpu_sc as plsc

N_NODES = 10000
N_PAD = 10240
E_TOTAL = 320000
D_IN = 128
NUM_HEADS = 8
HEAD_DIM = 16
FEAT = NUM_HEADS * HEAD_DIM

NUM_CORES = 2
NUM_SUBCORES = 16
NUM_WORKERS = NUM_CORES * NUM_SUBCORES
E_PER_W = E_TOTAL // NUM_WORKERS
B_EDGE = 80
NUM_BLOCKS = E_PER_W // B_EDGE
ROWS_PER_TILE = N_PAD // NUM_SUBCORES
ACC_CH = 128
NUM_CH = ROWS_PER_TILE // ACC_CH


def _proj_body(x_ref, w_ref, a_ref, xk_ref, ftab_ref):
    xk = jnp.dot(x_ref[...], w_ref[...], preferred_element_type=jnp.float32)
    xk_ref[...] = xk
    ftab_ref[...] = jnp.dot(xk, a_ref[...], preferred_element_type=jnp.float32)


def _project(x, w, a16):
    rb = 1000
    return pl.pallas_call(
        _proj_body,
        grid=(N_NODES // rb,),
        in_specs=[
            pl.BlockSpec((rb, D_IN), lambda i: (i, 0)),
            pl.BlockSpec((D_IN, FEAT), lambda i: (0, 0)),
            pl.BlockSpec((FEAT, 16), lambda i: (0, 0)),
        ],
        out_specs=[
            pl.BlockSpec((rb, FEAT), lambda i: (i, 0)),
            pl.BlockSpec((rb, 16), lambda i: (i, 0)),
        ],
        out_shape=[
            jax.ShapeDtypeStruct((N_NODES, FEAT), jnp.float32),
            jax.ShapeDtypeStruct((N_NODES, 16), jnp.float32),
        ],
    )(x, w, a16)


def _edge_body(xk, ftab, src, dst, acc_out, s_out,
               is0, it0, iu0, rx0, rt0, ru0, pr0,
               is1, it1, iu1, rx1, rt1, ru1, pr1,
               ic0, ic1, acc_sh, sacc_sh, sem_i, sem_g, sem_r0, sem_r1, sem_s0, sem_s1):
    cid = lax.axis_index("c")
    sid = lax.axis_index("s")
    wid = cid * NUM_SUBCORES + sid
    ebase = wid * E_PER_W
    LAST = NUM_BLOCKS - 1

    BUFS = ((is0, it0, iu0, rx0, rt0, ru0, pr0, sem_r0, ic0, sem_s0),
            (is1, it1, iu1, rx1, rt1, ru1, pr1, sem_r1, ic1, sem_s1))

    zero16 = jnp.zeros((16,), jnp.float32)

    def zrow(i, carry):
        for c in range(FEAT // 16):
            rx0[i, pl.ds(c * 16, 16)] = zero16
        pr0[i, :] = zero16
        return carry

    lax.fori_loop(0, B_EDGE, zrow, 0)
    base_row = sid * ROWS_PER_TILE
    for k in range(ROWS_PER_TILE // B_EDGE):
        pltpu.sync_copy(rx0, acc_sh.at[pl.ds(base_row + k * B_EDGE, B_EDGE), :])
        pltpu.sync_copy(pr0, sacc_sh.at[pl.ds(base_row + k * B_EDGE, B_EDGE), :])
    plsc.subcore_barrier()

    lanes = lax.iota(jnp.int32, 16)
    head_mask = lanes < NUM_HEADS

    def off(m):
        return pl.multiple_of(ebase + m * B_EDGE, 8)

    def start_idx(m, b):
        pltpu.async_copy(src.at[pl.ds(off(m), B_EDGE)], b[0], sem_i)
        pltpu.async_copy(dst.at[pl.ds(off(m), B_EDGE)], b[1], sem_i)

    def wait_idx(m, b):
        pltpu.make_async_copy(src.at[pl.ds(off(m), B_EDGE)], b[0], sem_i).wait()
        pltpu.make_async_copy(dst.at[pl.ds(off(m), B_EDGE)], b[1], sem_i).wait()

    def start_t2(b):
        pltpu.async_copy(dst.at[b[0]], b[2], sem_g)

    def wait_t2(b):
        pltpu.make_async_copy(dst.at[b[0]], b[2], sem_g).wait()

    def start_rows_xt(b):
        pltpu.async_copy(xk.at[b[0]], b[3], b[7])
        pltpu.async_copy(ftab.at[b[1]], b[4], b[7])

    def start_rows_u(b):
        pltpu.async_copy(ftab.at[b[2]], b[5], b[7])

    def wait_rows(b):
        pltpu.make_async_copy(xk.at[b[0]], b[3], b[7]).wait()
        pltpu.make_async_copy(ftab.at[b[1]], b[4], b[7]).wait()
        pltpu.make_async_copy(ftab.at[b[2]], b[5], b[7]).wait()

    def compute(b):
        rows_x, rows_t, rows_u, prow = b[3], b[4], b[5], b[6]

        def edge2(j, ecarry):
            e0 = 2 * j
            e1 = e0 + 1
            z0 = rows_t[e0, :] + rows_u[e0, :]
            z1 = rows_t[e1, :] + rows_u[e1, :]
            p0 = jnp.where(head_mask, jnp.exp(jnp.maximum(z0, 0.2 * z0)), 0.0)
            p1 = jnp.where(head_mask, jnp.exp(jnp.maximum(z1, 0.2 * z1)), 0.0)
            prow[e0, :] = p0
            prow[e1, :] = p1
            l0 = [rows_x[e0, pl.ds(h * HEAD_DIM, HEAD_DIM)] for h in range(NUM_HEADS)]
            l1 = [rows_x[e1, pl.ds(h * HEAD_DIM, HEAD_DIM)] for h in range(NUM_HEADS)]
            m0 = [l0[h] * p0[h] for h in range(NUM_HEADS)]
            m1 = [l1[h] * p1[h] for h in range(NUM_HEADS)]
            for h in range(NUM_HEADS):
                rows_x[e0, pl.ds(h * HEAD_DIM, HEAD_DIM)] = m0[h]
            for h in range(NUM_HEADS):
                rows_x[e1, pl.ds(h * HEAD_DIM, HEAD_DIM)] = m1[h]
            return ecarry

        lax.fori_loop(0, B_EDGE // 2, edge2, 0)

    def scatter_start(b):
        for c in range(B_EDGE // 16):
            b[8][pl.ds(c * 16, 16)] = b[1][pl.ds(c * 16, 16)]
        pltpu.async_copy(b[3], acc_sh.at[b[8]], b[9], add=True)
        pltpu.async_copy(b[6], sacc_sh.at[b[8]], b[9], add=True)

    def wait_scatter(b):
        pltpu.make_async_copy(b[3], acc_sh.at[b[8]], b[9]).wait()
        pltpu.make_async_copy(b[6], sacc_sh.at[b[8]], b[9]).wait()

    def process(m, par):
        cur, nxt = BUFS[par], BUFS[1 - par]

        @pl.when(jnp.logical_and(m >= 1, m + 1 <= LAST))
        def _():
            wait_scatter(nxt)

        @pl.when(m + 1 <= LAST)
        def _():
            wait_idx(m + 1, nxt)
            start_t2(nxt)
            start_rows_xt(nxt)

        wait_rows(cur)
        compute(cur)
        scatter_start(cur)

        @pl.when(m + 2 <= LAST)
        def _():
            start_idx(m + 2, cur)

        @pl.when(m + 1 <= LAST)
        def _():
            wait_t2(nxt)
            start_rows_u(nxt)

    start_idx(0, BUFS[0])
    wait_idx(0, BUFS[0])
    start_t2(BUFS[0])
    start_rows_xt(BUFS[0])
    start_idx(1, BUFS[1])
    wait_t2(BUFS[0])
    start_rows_u(BUFS[0])

    def outer(i, carry):
        process(2 * i, 0)
        process(2 * i + 1, 1)
        return carry

    lax.fori_loop(0, NUM_BLOCKS // 2, outer, 0)
    process(LAST, 0)
    wait_scatter(BUFS[1])
    wait_scatter(BUFS[0])
    plsc.subcore_barrier()

    for k in range(NUM_CH):
        r0 = base_row + k * ACC_CH
        pltpu.sync_copy(acc_sh.at[pl.ds(r0, ACC_CH), :],
                        acc_out.at[cid, pl.ds(r0, ACC_CH), :])
        pltpu.sync_copy(sacc_sh.at[pl.ds(r0, ACC_CH), :],
                        s_out.at[cid, pl.ds(r0, ACC_CH), :])

_edge_pass = functools.partial(
    pl.kernel,
    out_type=[
        jax.ShapeDtypeStruct((NUM_CORES, N_PAD, FEAT), jnp.float32),
        jax.ShapeDtypeStruct((NUM_CORES, N_PAD, 16), jnp.float32),
    ],
    mesh=plsc.VectorSubcoreMesh(
        core_axis_name="c", subcore_axis_name="s",
        num_cores=NUM_CORES, num_subcores=NUM_SUBCORES),
    compiler_params=pltpu.CompilerParams(use_tc_tiling_on_sc=False),
    scratch_types=[
        pltpu.VMEM((B_EDGE,), jnp.int32),
        pltpu.VMEM((B_EDGE,), jnp.int32),
        pltpu.VMEM((B_EDGE,), jnp.int32),
        pltpu.VMEM((B_EDGE, FEAT), jnp.float32),
        pltpu.VMEM((B_EDGE, 16), jnp.float32),
        pltpu.VMEM((B_EDGE, 16), jnp.float32),
        pltpu.VMEM((B_EDGE, 16), jnp.float32),
        pltpu.VMEM((B_EDGE,), jnp.int32),
        pltpu.VMEM((B_EDGE,), jnp.int32),
        pltpu.VMEM((B_EDGE,), jnp.int32),
        pltpu.VMEM((B_EDGE, FEAT), jnp.float32),
        pltpu.VMEM((B_EDGE, 16), jnp.float32),
        pltpu.VMEM((B_EDGE, 16), jnp.float32),
        pltpu.VMEM((B_EDGE, 16), jnp.float32),
        pltpu.VMEM((B_EDGE,), jnp.int32),
        pltpu.VMEM((B_EDGE,), jnp.int32),
        pltpu.VMEM_SHARED((N_PAD, FEAT), jnp.float32),
        pltpu.VMEM_SHARED((N_PAD, 16), jnp.float32),
        pltpu.SemaphoreType.DMA,
        pltpu.SemaphoreType.DMA,
        pltpu.SemaphoreType.DMA,
        pltpu.SemaphoreType.DMA,
        pltpu.SemaphoreType.DMA,
        pltpu.SemaphoreType.DMA,
    ],
)(_edge_body)


def _final_body(acc_ref, s_ref, rep_ref, bias_ref, o_ref):
    num = acc_ref[0] + acc_ref[1]
    den = s_ref[0] + s_ref[1]
    den_rep = jnp.dot(den, rep_ref[...], preferred_element_type=jnp.float32)
    y = num / (den_rep + 1e-7) + bias_ref[...]
    o_ref[...] = jnp.where(y > 0, y, jnp.exp(jnp.minimum(y, 0.0)) - 1.0)


def _finalize(acc, s, rep, bias2d):
    rb = 1000
    return pl.pallas_call(
        _final_body,
        grid=(N_NODES // rb,),
        in_specs=[
            pl.BlockSpec((NUM_CORES, rb, FEAT), lambda i: (0, i, 0)),
            pl.BlockSpec((NUM_CORES, rb, 16), lambda i: (0, i, 0)),
            pl.BlockSpec((16, FEAT), lambda i: (0, 0)),
            pl.BlockSpec((1, FEAT), lambda i: (0, 0)),
        ],
        out_specs=pl.BlockSpec((rb, FEAT), lambda i: (i, 0)),
        out_shape=jax.ShapeDtypeStruct((N_NODES, FEAT), jnp.float32),
    )(acc, s, rep, bias2d)


def kernel(x, edges, kernel, kernel_attention1, kernel_attention2, bias, training):
    del kernel_attention2, training
    src = edges[:, 0].astype(jnp.int32)
    dst = edges[:, 1].astype(jnp.int32)

    ka1 = kernel_attention1[0].astype(jnp.float32)
    a8 = (ka1[:, :, None] * jnp.eye(NUM_HEADS, dtype=jnp.float32)[:, None, :])
    a16 = jnp.concatenate(
        [a8.reshape(FEAT, NUM_HEADS),
         jnp.zeros((FEAT, NUM_HEADS), jnp.float32)], axis=1)

    rep = jnp.concatenate(
        [jnp.kron(jnp.eye(NUM_HEADS, dtype=jnp.float32),
                  jnp.ones((1, HEAD_DIM), jnp.float32)),
         jnp.zeros((NUM_HEADS, FEAT), jnp.float32)], axis=0)

    xk, ftab = _project(x, kernel.astype(jnp.float32), a16)
    acc, s = _edge_pass(xk, ftab, src, dst)
    return _finalize(acc, s, rep, bias.reshape(1, FEAT).astype(jnp.float32))


# speedup vs baseline: 148.2171x; 1.1488x over previous
import functools

import jax
import jax.numpy as jnp
from jax import lax
from jax.experimental import pallas as pl
from jax.experimental.pallas import tpu as pltpu
from jax.experimental.pallas import tpu_sc as plsc

N_NODES = 10000
N_PAD = 10240
E_TOTAL = 320000
D_IN = 128
NUM_HEADS = 8
HEAD_DIM = 16
FEAT = NUM_HEADS * HEAD_DIM

NUM_CORES = 2
NUM_SUBCORES = 16
NUM_WORKERS = NUM_CORES * NUM_SUBCORES
E_PER_W = E_TOTAL // NUM_WORKERS
B_EDGE = 80
NUM_BLOCKS = E_PER_W // B_EDGE
ROWS_PER_TILE = N_PAD // NUM_SUBCORES
ACC_CH = 128
NUM_CH = ROWS_PER_TILE // ACC_CH


def _proj_body(x_ref, w_ref, a_ref, xk_ref, ftab_ref):
    xk = jnp.dot(x_ref[...], w_ref[...], preferred_element_type=jnp.float32)
    xk_ref[...] = xk
    ftab_ref[...] = jnp.dot(xk, a_ref[...], preferred_element_type=jnp.float32)


def _project(x, w, a16):
    rb = 1000
    return pl.pallas_call(
        _proj_body,
        grid=(N_NODES // rb,),
        in_specs=[
            pl.BlockSpec((rb, D_IN), lambda i: (i, 0)),
            pl.BlockSpec((D_IN, FEAT), lambda i: (0, 0)),
            pl.BlockSpec((FEAT, 16), lambda i: (0, 0)),
        ],
        out_specs=[
            pl.BlockSpec((rb, FEAT), lambda i: (i, 0)),
            pl.BlockSpec((rb, 16), lambda i: (i, 0)),
        ],
        out_shape=[
            jax.ShapeDtypeStruct((N_NODES, FEAT), jnp.float32),
            jax.ShapeDtypeStruct((N_NODES, 16), jnp.float32),
        ],
    )(x, w, a16)


def _edge_body(xk, ftab, src, dst, acc_out, s_out,
               is0, it0, iu0, rx0, rt0, ru0, pr0,
               is1, it1, iu1, rx1, rt1, ru1, pr1,
               ic0, ic1, acc_sh, sacc_sh, sem_i, sem_g, sem_r0, sem_r1, sem_s0, sem_s1):
    cid = lax.axis_index("c")
    sid = lax.axis_index("s")
    wid = cid * NUM_SUBCORES + sid
    ebase = wid * E_PER_W
    LAST = NUM_BLOCKS - 1

    BUFS = ((is0, it0, iu0, rx0, rt0, ru0, pr0, sem_r0, ic0, sem_s0),
            (is1, it1, iu1, rx1, rt1, ru1, pr1, sem_r1, ic1, sem_s1))

    zero16 = jnp.zeros((16,), jnp.float32)

    def zrow(i, carry):
        for c in range(FEAT // 16):
            rx0[i, pl.ds(c * 16, 16)] = zero16
        pr0[i, :] = zero16
        return carry

    lax.fori_loop(0, B_EDGE, zrow, 0)
    base_row = sid * ROWS_PER_TILE
    for k in range(ROWS_PER_TILE // B_EDGE):
        pltpu.sync_copy(rx0, acc_sh.at[pl.ds(base_row + k * B_EDGE, B_EDGE), :])
        pltpu.sync_copy(pr0, sacc_sh.at[pl.ds(base_row + k * B_EDGE, B_EDGE), :])
    plsc.subcore_barrier()

    lanes = lax.iota(jnp.int32, 16)
    head_mask = lanes < NUM_HEADS

    def off(m):
        return pl.multiple_of(ebase + m * B_EDGE, 8)

    def start_idx(m, b):
        pltpu.async_copy(src.at[pl.ds(off(m), B_EDGE)], b[0], sem_i)
        pltpu.async_copy(dst.at[pl.ds(off(m), B_EDGE)], b[1], sem_i)

    def wait_idx(m, b):
        pltpu.make_async_copy(src.at[pl.ds(off(m), B_EDGE)], b[0], sem_i).wait()
        pltpu.make_async_copy(dst.at[pl.ds(off(m), B_EDGE)], b[1], sem_i).wait()

    def start_t2(b):
        pltpu.async_copy(dst.at[b[0]], b[2], sem_g)

    def wait_t2(b):
        pltpu.make_async_copy(dst.at[b[0]], b[2], sem_g).wait()

    def start_rows_xt(b):
        pltpu.async_copy(xk.at[b[0]], b[3], b[7])
        pltpu.async_copy(ftab.at[b[1]], b[4], b[7])

    def start_rows_u(b):
        pltpu.async_copy(ftab.at[b[2]], b[5], b[7])

    def wait_rows(b):
        pltpu.make_async_copy(xk.at[b[0]], b[3], b[7]).wait()
        pltpu.make_async_copy(ftab.at[b[1]], b[4], b[7]).wait()
        pltpu.make_async_copy(ftab.at[b[2]], b[5], b[7]).wait()

    def compute(b):
        rows_x, rows_t, rows_u, prow = b[3], b[4], b[5], b[6]

        def edge4(j, ecarry):
            es = [4 * j + q for q in range(4)]
            zs = [rows_t[e, :] + rows_u[e, :] for e in es]
            ps = [jnp.where(head_mask,
                            jnp.exp(jnp.maximum(z, 0.2 * z)), 0.0) for z in zs]
            for e, p in zip(es, ps):
                prow[e, :] = p
            ls = [[rows_x[e, pl.ds(h * HEAD_DIM, HEAD_DIM)]
                   for h in range(NUM_HEADS)] for e in es]
            ms = [[ls[q][h] * ps[q][h] for h in range(NUM_HEADS)]
                  for q in range(4)]
            for q, e in enumerate(es):
                for h in range(NUM_HEADS):
                    rows_x[e, pl.ds(h * HEAD_DIM, HEAD_DIM)] = ms[q][h]
            return ecarry

        lax.fori_loop(0, B_EDGE // 4, edge4, 0)

    def scatter_start(b):
        for c in range(B_EDGE // 16):
            b[8][pl.ds(c * 16, 16)] = b[1][pl.ds(c * 16, 16)]
        pltpu.async_copy(b[3], acc_sh.at[b[8]], b[9], add=True)
        pltpu.async_copy(b[6], sacc_sh.at[b[8]], b[9], add=True)

    def wait_scatter(b):
        pltpu.make_async_copy(b[3], acc_sh.at[b[8]], b[9]).wait()
        pltpu.make_async_copy(b[6], sacc_sh.at[b[8]], b[9]).wait()

    def process(m, par):
        cur, nxt = BUFS[par], BUFS[1 - par]

        @pl.when(jnp.logical_and(m >= 1, m + 1 <= LAST))
        def _():
            wait_scatter(nxt)

        @pl.when(m + 1 <= LAST)
        def _():
            wait_idx(m + 1, nxt)
            start_t2(nxt)
            start_rows_xt(nxt)

        wait_rows(cur)
        compute(cur)
        scatter_start(cur)

        @pl.when(m + 2 <= LAST)
        def _():
            start_idx(m + 2, cur)

        @pl.when(m + 1 <= LAST)
        def _():
            wait_t2(nxt)
            start_rows_u(nxt)

    start_idx(0, BUFS[0])
    wait_idx(0, BUFS[0])
    start_t2(BUFS[0])
    start_rows_xt(BUFS[0])
    start_idx(1, BUFS[1])
    wait_t2(BUFS[0])
    start_rows_u(BUFS[0])

    def outer(i, carry):
        process(2 * i, 0)
        process(2 * i + 1, 1)
        return carry

    lax.fori_loop(0, NUM_BLOCKS // 2, outer, 0)
    process(LAST, 0)
    wait_scatter(BUFS[1])
    wait_scatter(BUFS[0])
    plsc.subcore_barrier()

    for k in range(NUM_CH):
        r0 = base_row + k * ACC_CH
        pltpu.sync_copy(acc_sh.at[pl.ds(r0, ACC_CH), :],
                        acc_out.at[cid, pl.ds(r0, ACC_CH), :])
        pltpu.sync_copy(sacc_sh.at[pl.ds(r0, ACC_CH), :],
                        s_out.at[cid, pl.ds(r0, ACC_CH), :])

_edge_pass = functools.partial(
    pl.kernel,
    out_type=[
        jax.ShapeDtypeStruct((NUM_CORES, N_PAD, FEAT), jnp.float32),
        jax.ShapeDtypeStruct((NUM_CORES, N_PAD, 16), jnp.float32),
    ],
    mesh=plsc.VectorSubcoreMesh(
        core_axis_name="c", subcore_axis_name="s",
        num_cores=NUM_CORES, num_subcores=NUM_SUBCORES),
    compiler_params=pltpu.CompilerParams(use_tc_tiling_on_sc=False),
    scratch_types=[
        pltpu.VMEM((B_EDGE,), jnp.int32),
        pltpu.VMEM((B_EDGE,), jnp.int32),
        pltpu.VMEM((B_EDGE,), jnp.int32),
        pltpu.VMEM((B_EDGE, FEAT), jnp.float32),
        pltpu.VMEM((B_EDGE, 16), jnp.float32),
        pltpu.VMEM((B_EDGE, 16), jnp.float32),
        pltpu.VMEM((B_EDGE, 16), jnp.float32),
        pltpu.VMEM((B_EDGE,), jnp.int32),
        pltpu.VMEM((B_EDGE,), jnp.int32),
        pltpu.VMEM((B_EDGE,), jnp.int32),
        pltpu.VMEM((B_EDGE, FEAT), jnp.float32),
        pltpu.VMEM((B_EDGE, 16), jnp.float32),
        pltpu.VMEM((B_EDGE, 16), jnp.float32),
        pltpu.VMEM((B_EDGE, 16), jnp.float32),
        pltpu.VMEM((B_EDGE,), jnp.int32),
        pltpu.VMEM((B_EDGE,), jnp.int32),
        pltpu.VMEM_SHARED((N_PAD, FEAT), jnp.float32),
        pltpu.VMEM_SHARED((N_PAD, 16), jnp.float32),
        pltpu.SemaphoreType.DMA,
        pltpu.SemaphoreType.DMA,
        pltpu.SemaphoreType.DMA,
        pltpu.SemaphoreType.DMA,
        pltpu.SemaphoreType.DMA,
        pltpu.SemaphoreType.DMA,
    ],
)(_edge_body)


def _final_body(acc_ref, s_ref, rep_ref, bias_ref, o_ref):
    num = acc_ref[0] + acc_ref[1]
    den = s_ref[0] + s_ref[1]
    den_rep = jnp.dot(den, rep_ref[...], preferred_element_type=jnp.float32)
    y = num / (den_rep + 1e-7) + bias_ref[...]
    o_ref[...] = jnp.where(y > 0, y, jnp.exp(jnp.minimum(y, 0.0)) - 1.0)


def _finalize(acc, s, rep, bias2d):
    rb = 1000
    return pl.pallas_call(
        _final_body,
        grid=(N_NODES // rb,),
        in_specs=[
            pl.BlockSpec((NUM_CORES, rb, FEAT), lambda i: (0, i, 0)),
            pl.BlockSpec((NUM_CORES, rb, 16), lambda i: (0, i, 0)),
            pl.BlockSpec((16, FEAT), lambda i: (0, 0)),
            pl.BlockSpec((1, FEAT), lambda i: (0, 0)),
        ],
        out_specs=pl.BlockSpec((rb, FEAT), lambda i: (i, 0)),
        out_shape=jax.ShapeDtypeStruct((N_NODES, FEAT), jnp.float32),
    )(acc, s, rep, bias2d)


def kernel(x, edges, kernel, kernel_attention1, kernel_attention2, bias, training):
    del kernel_attention2, training
    src = edges[:, 0].astype(jnp.int32)
    dst = edges[:, 1].astype(jnp.int32)

    ka1 = kernel_attention1[0].astype(jnp.float32)
    a8 = (ka1[:, :, None] * jnp.eye(NUM_HEADS, dtype=jnp.float32)[:, None, :])
    a16 = jnp.concatenate(
        [a8.reshape(FEAT, NUM_HEADS),
         jnp.zeros((FEAT, NUM_HEADS), jnp.float32)], axis=1)

    rep = jnp.concatenate(
        [jnp.kron(jnp.eye(NUM_HEADS, dtype=jnp.float32),
                  jnp.ones((1, HEAD_DIM), jnp.float32)),
         jnp.zeros((NUM_HEADS, FEAT), jnp.float32)], axis=0)

    xk, ftab = _project(x, kernel.astype(jnp.float32), a16)
    acc, s = _edge_pass(xk, ftab, src, dst)
    return _finalize(acc, s, rep, bias.reshape(1, FEAT).astype(jnp.float32))
